# Initial kernel scaffold; baseline (speedup 1.0000x reference)
import functools

import jax
import jax.numpy as jnp
from jax import lax
from jax.experimental import pallas as pl
from jax.experimental.pallas import tpu as pltpu
from jax.experimental.pallas import tpu_sc as plsc

NC = 2
NS = 16
NW = NC * NS
CR = 56


def _sc_hist(npad, rw):
  mesh = plsc.VectorSubcoreMesh(core_axis_name="c", subcore_axis_name="s")
  nslice = npad // NS
  nchunks = rw // CR

  @functools.partial(
      pl.kernel,
      out_type=jax.ShapeDtypeStruct((NC, npad), jnp.float32),
      mesh=mesh,
      scratch_types=[
          pltpu.VMEM((CR, 128), jnp.int32),
          pltpu.VMEM((CR, 128), jnp.float32),
          pltpu.VMEM_SHARED((npad,), jnp.float32),
      ],
  )
  def k(col_hbm, zeros_hbm, ones_hbm, out_hbm, idx_v, ones_v, hist_sh):
    cid = lax.axis_index("c")
    sid = lax.axis_index("s")
    wid = sid * NC + cid
    pltpu.sync_copy(zeros_hbm.at[pl.ds(sid * nslice, nslice)],
                    hist_sh.at[pl.ds(sid * nslice, nslice)])
    pltpu.sync_copy(ones_hbm, ones_v)
    plsc.subcore_barrier()

    @pl.loop(0, nchunks)
    def _(t):
      base = wid * rw + t * CR
      pltpu.sync_copy(col_hbm.at[pl.ds(base, CR)], idx_v)
      pltpu.sync_copy(ones_v, hist_sh.at[idx_v], add=True)

    plsc.subcore_barrier()
    pltpu.sync_copy(hist_sh.at[pl.ds(sid * nslice, nslice)],
                    out_hbm.at[cid, pl.ds(sid * nslice, nslice)])

  return k


def _sc_scatter(npad, rw):
  mesh = plsc.VectorSubcoreMesh(core_axis_name="c", subcore_axis_name="s")
  nslice = npad // NS
  nchunks = rw // CR

  @functools.partial(
      pl.kernel,
      out_type=jax.ShapeDtypeStruct((NC, npad, 2), jnp.float32),
      mesh=mesh,
      scratch_types=[
          pltpu.VMEM((CR, 128), jnp.int32),
          pltpu.VMEM((CR, 128), jnp.int32),
          pltpu.VMEM((CR, 128, 2), jnp.float32),
          pltpu.VMEM_SHARED((npad, 2), jnp.float32),
      ],
  )
  def k(row_hbm, col_hbm, s_hbm, zeros2_hbm, out_hbm,
        row_v, col_v, val_v, acc_sh):
    cid = lax.axis_index("c")
    sid = lax.axis_index("s")
    wid = sid * NC + cid
    pltpu.sync_copy(zeros2_hbm.at[pl.ds(sid * nslice, nslice)],
                    acc_sh.at[pl.ds(sid * nslice, nslice)])
    plsc.subcore_barrier()

    @pl.loop(0, nchunks)
    def _(t):
      base = wid * rw + t * CR
      pltpu.sync_copy(row_hbm.at[pl.ds(base, CR)], row_v)
      pltpu.sync_copy(col_hbm.at[pl.ds(base, CR)], col_v)
      pltpu.sync_copy(s_hbm.at[row_v], val_v)
      pltpu.sync_copy(val_v, acc_sh.at[col_v], add=True)

    plsc.subcore_barrier()
    pltpu.sync_copy(acc_sh.at[pl.ds(sid * nslice, nslice)],
                    out_hbm.at[cid, pl.ds(sid * nslice, nslice)])

  return k


def _tc_prep(npad, blk):
  grid = npad // blk

  def body(h_ref, x_ref, s_ref, d_ref):
    h = h_ref[...]
    deg = h[0] + h[1] + 1.0
    dinv = lax.rsqrt(deg)
    s_ref[...] = x_ref[...] * dinv
    d_ref[...] = dinv

  return pl.pallas_call(
      body,
      grid=(grid,),
      in_specs=[
          pl.BlockSpec((NC, blk, 1), lambda i: (0, i, 0)),
          pl.BlockSpec((blk, 2), lambda i: (i, 0)),
      ],
      out_specs=[
          pl.BlockSpec((blk, 2), lambda i: (i, 0)),
          pl.BlockSpec((blk, 1), lambda i: (i, 0)),
      ],
      out_shape=[
          jax.ShapeDtypeStruct((npad, 2), jnp.float32),
          jax.ShapeDtypeStruct((npad, 1), jnp.float32),
      ],
  )


def _tc_final(n, blk, hid, res, out_dim):
  grid = n // blk

  def body(acc_ref, s_ref, d_ref, esn_ref, wg_ref, bg_ref, lnw_ref, lnb_ref,
           w1a_ref, w1b_ref, b1_ref, w2_ref, b2_ref, o_ref, g_acc):
    i = pl.program_id(0)

    @pl.when(i == 0)
    def _():
      g_acc[...] = jnp.zeros_like(g_acc)

    acc = acc_ref[...]
    a = (acc[0] + acc[1] + s_ref[...]) * d_ref[...]
    wg = wg_ref[...]
    h = a[:, 0:1] * wg[0:1, :] + a[:, 1:2] * wg[1:2, :] + bg_ref[...]
    h = jnp.maximum(h, 0.0)
    mu = jnp.mean(h, axis=1, keepdims=True)
    hc = h - mu
    var = jnp.mean(hc * hc, axis=1, keepdims=True)
    normed = hc * lax.rsqrt(var + 1e-5) * lnw_ref[...] + lnb_ref[...]
    g_acc[...] += jnp.sum(normed, axis=0, keepdims=True)

    @pl.when(i == grid - 1)
    def _():
      g = g_acc[...]
      z = (jnp.dot(g, w1a_ref[...], preferred_element_type=jnp.float32)
           + jnp.dot(esn_ref[...], w1b_ref[...],
                     preferred_element_type=jnp.float32)
           + b1_ref[...])
      z = jnp.maximum(z, 0.0)
      logits = jnp.dot(z, w2_ref[...],
                       preferred_element_type=jnp.float32) + b2_ref[...]
      m = jnp.max(logits, axis=1, keepdims=True)
      lse = jnp.log(jnp.sum(jnp.exp(logits - m), axis=1, keepdims=True)) + m
      o_ref[...] = logits - lse

  z0 = lambda i: (0, 0)
  return pl.pallas_call(
      body,
      grid=(grid,),
      in_specs=[
          pl.BlockSpec((NC, blk, 2), lambda i: (0, i, 0)),
          pl.BlockSpec((blk, 2), lambda i: (i, 0)),
          pl.BlockSpec((blk, 1), lambda i: (i, 0)),
          pl.BlockSpec((1, res), z0),
          pl.BlockSpec((2, hid), z0),
          pl.BlockSpec((1, hid), z0),
          pl.BlockSpec((1, hid), z0),
          pl.BlockSpec((1, hid), z0),
          pl.BlockSpec((hid, hid), z0),
          pl.BlockSpec((res, hid), z0),
          pl.BlockSpec((1, hid), z0),
          pl.BlockSpec((hid, out_dim), z0),
          pl.BlockSpec((1, out_dim), z0),
      ],
      out_specs=pl.BlockSpec((1, out_dim), z0),
      out_shape=jax.ShapeDtypeStruct((1, out_dim), jnp.float32),
      scratch_shapes=[pltpu.VMEM((1, 128), jnp.float32)],
  )


def kernel(node_features, edge_index, esn_state, W_gcn, b_gcn, ln_w, ln_b,
           W1, b1, W2, b2):
  n, _ = node_features.shape
  e = edge_index.shape[1]
  hid = W_gcn.shape[1]
  res = esn_state.shape[1]
  out_dim = W2.shape[1]

  npad = ((n + 1 + 2047) // 2048) * 2048
  rows = -(-e // 128)
  rw = -(-rows // (NW * CR)) * CR
  epad = NW * rw * 128

  row = jnp.concatenate(
      [edge_index[0], jnp.zeros((epad - e,), jnp.int32)]).reshape(NW * rw, 128)
  col = jnp.concatenate(
      [edge_index[1], jnp.full((epad - e,), n, jnp.int32)]).reshape(NW * rw, 128)
  x_pad = jnp.pad(node_features, ((0, npad - n), (0, 0)))
  zeros1 = jnp.zeros((npad,), jnp.float32)
  zeros2 = jnp.zeros((npad, 2), jnp.float32)
  ones2d = jnp.ones((CR, 128), jnp.float32)

  hist = _sc_hist(npad, rw)(col, zeros1, ones2d)
  s, dinv = _tc_prep(npad, 2048)(hist.reshape(NC, npad, 1), x_pad)
  acc = _sc_scatter(npad, rw)(row, col, s, zeros2)

  blk = 4000
  return _tc_final(n, blk, hid, res, out_dim)(
      acc, s, dinv, esn_state,
      W_gcn, b_gcn.reshape(1, hid), ln_w.reshape(1, hid),
      ln_b.reshape(1, hid), W1[:hid], W1[hid:], b1.reshape(1, hid),
      W2, b2.reshape(1, out_dim))


# speedup vs baseline: 37.7282x; 37.7282x over previous
import functools

import jax
import jax.numpy as jnp
from jax import lax
from jax.experimental import pallas as pl
from jax.experimental.pallas import tpu as pltpu
from jax.experimental.pallas import tpu_sc as plsc

NC = 2
NS = 16
NW = NC * NS
CR = 56


def _sc_hist(npad, rw):
  mesh = plsc.VectorSubcoreMesh(core_axis_name="c", subcore_axis_name="s")
  nslice = npad // NS
  nchunks = rw // CR

  @functools.partial(
      pl.kernel,
      out_type=jax.ShapeDtypeStruct((NC, npad), jnp.float32),
      mesh=mesh,
      compiler_params=pltpu.CompilerParams(use_tc_tiling_on_sc=False),
      scratch_types=[
          pltpu.VMEM((CR, 128), jnp.int32),
          pltpu.VMEM((1, 128), jnp.float32),
          pltpu.VMEM_SHARED((npad,), jnp.float32),
      ],
  )
  def k(col_hbm, zeros_hbm, ones_hbm, out_hbm, idx_v, ones_v, hist_sh):
    cid = lax.axis_index("c")
    sid = lax.axis_index("s")
    wid = sid * NC + cid
    pltpu.sync_copy(zeros_hbm.at[pl.ds(sid * nslice, nslice)],
                    hist_sh.at[pl.ds(sid * nslice, nslice)])
    pltpu.sync_copy(ones_hbm, ones_v)
    plsc.subcore_barrier()

    @pl.loop(0, nchunks)
    def _(t):
      base = wid * rw + t * CR
      pltpu.sync_copy(col_hbm.at[pl.ds(base, CR)], idx_v)

      @pl.loop(0, CR)
      def _(j):
        pltpu.sync_copy(ones_v.at[0], hist_sh.at[idx_v.at[j]], add=True)

    plsc.subcore_barrier()
    pltpu.sync_copy(hist_sh.at[pl.ds(sid * nslice, nslice)],
                    out_hbm.at[cid, pl.ds(sid * nslice, nslice)])

  return k


def _sc_scatter(npad, rw):
  mesh = plsc.VectorSubcoreMesh(core_axis_name="c", subcore_axis_name="s")
  nslice = npad // NS
  nchunks = rw // CR

  @functools.partial(
      pl.kernel,
      out_type=jax.ShapeDtypeStruct((NC, npad, 2), jnp.float32),
      mesh=mesh,
      compiler_params=pltpu.CompilerParams(use_tc_tiling_on_sc=False),
      scratch_types=[
          pltpu.VMEM((CR, 128), jnp.int32),
          pltpu.VMEM((CR, 128), jnp.int32),
          pltpu.VMEM((CR, 128, 2), jnp.float32),
          pltpu.VMEM_SHARED((npad, 2), jnp.float32),
      ],
  )
  def k(row_hbm, col_hbm, s_hbm, zeros2_hbm, out_hbm,
        row_v, col_v, val_v, acc_sh):
    cid = lax.axis_index("c")
    sid = lax.axis_index("s")
    wid = sid * NC + cid
    pltpu.sync_copy(zeros2_hbm.at[pl.ds(sid * nslice, nslice)],
                    acc_sh.at[pl.ds(sid * nslice, nslice)])
    plsc.subcore_barrier()

    @pl.loop(0, nchunks)
    def _(t):
      base = wid * rw + t * CR
      pltpu.sync_copy(row_hbm.at[pl.ds(base, CR)], row_v)
      pltpu.sync_copy(col_hbm.at[pl.ds(base, CR)], col_v)

      @pl.loop(0, CR)
      def _(j):
        pltpu.sync_copy(s_hbm.at[row_v.at[j]], val_v.at[j])
        pltpu.sync_copy(val_v.at[j], acc_sh.at[col_v.at[j]], add=True)

    plsc.subcore_barrier()
    pltpu.sync_copy(acc_sh.at[pl.ds(sid * nslice, nslice)],
                    out_hbm.at[cid, pl.ds(sid * nslice, nslice)])

  return k


def _tc_prep(npad, blk):
  grid = npad // blk

  def body(h_ref, x_ref, s_ref, d_ref):
    h = h_ref[...]
    deg = h[0] + h[1] + 1.0
    dinv = lax.rsqrt(deg)
    s_ref[...] = x_ref[...] * dinv
    d_ref[...] = dinv

  return pl.pallas_call(
      body,
      grid=(grid,),
      in_specs=[
          pl.BlockSpec((NC, blk, 1), lambda i: (0, i, 0)),
          pl.BlockSpec((blk, 2), lambda i: (i, 0)),
      ],
      out_specs=[
          pl.BlockSpec((blk, 2), lambda i: (i, 0)),
          pl.BlockSpec((blk, 1), lambda i: (i, 0)),
      ],
      out_shape=[
          jax.ShapeDtypeStruct((npad, 2), jnp.float32),
          jax.ShapeDtypeStruct((npad, 1), jnp.float32),
      ],
  )


def _tc_final(n, blk, hid, res, out_dim):
  grid = n // blk

  def body(acc_ref, s_ref, d_ref, esn_ref, wg_ref, bg_ref, lnw_ref, lnb_ref,
           w1a_ref, w1b_ref, b1_ref, w2_ref, b2_ref, o_ref, g_acc):
    i = pl.program_id(0)

    @pl.when(i == 0)
    def _():
      g_acc[...] = jnp.zeros_like(g_acc)

    acc = acc_ref[...]
    a = (acc[0] + acc[1] + s_ref[...]) * d_ref[...]
    wg = wg_ref[...]
    h = a[:, 0:1] * wg[0:1, :] + a[:, 1:2] * wg[1:2, :] + bg_ref[...]
    h = jnp.maximum(h, 0.0)
    mu = jnp.mean(h, axis=1, keepdims=True)
    hc = h - mu
    var = jnp.mean(hc * hc, axis=1, keepdims=True)
    normed = hc * lax.rsqrt(var + 1e-5) * lnw_ref[...] + lnb_ref[...]
    g_acc[...] += jnp.sum(normed, axis=0, keepdims=True)

    @pl.when(i == grid - 1)
    def _():
      g = g_acc[...]
      z = (jnp.dot(g, w1a_ref[...], preferred_element_type=jnp.float32)
           + jnp.dot(esn_ref[...], w1b_ref[...],
                     preferred_element_type=jnp.float32)
           + b1_ref[...])
      z = jnp.maximum(z, 0.0)
      logits = jnp.dot(z, w2_ref[...],
                       preferred_element_type=jnp.float32) + b2_ref[...]
      m = jnp.max(logits, axis=1, keepdims=True)
      lse = jnp.log(jnp.sum(jnp.exp(logits - m), axis=1, keepdims=True)) + m
      o_ref[...] = logits - lse

  z0 = lambda i: (0, 0)
  return pl.pallas_call(
      body,
      grid=(grid,),
      in_specs=[
          pl.BlockSpec((NC, blk, 2), lambda i: (0, i, 0)),
          pl.BlockSpec((blk, 2), lambda i: (i, 0)),
          pl.BlockSpec((blk, 1), lambda i: (i, 0)),
          pl.BlockSpec((1, res), z0),
          pl.BlockSpec((2, hid), z0),
          pl.BlockSpec((1, hid), z0),
          pl.BlockSpec((1, hid), z0),
          pl.BlockSpec((1, hid), z0),
          pl.BlockSpec((hid, hid), z0),
          pl.BlockSpec((res, hid), z0),
          pl.BlockSpec((1, hid), z0),
          pl.BlockSpec((hid, out_dim), z0),
          pl.BlockSpec((1, out_dim), z0),
      ],
      out_specs=pl.BlockSpec((1, out_dim), z0),
      out_shape=jax.ShapeDtypeStruct((1, out_dim), jnp.float32),
      scratch_shapes=[pltpu.VMEM((1, 128), jnp.float32)],
  )


def kernel(node_features, edge_index, esn_state, W_gcn, b_gcn, ln_w, ln_b,
           W1, b1, W2, b2):
  n, _ = node_features.shape
  e = edge_index.shape[1]
  hid = W_gcn.shape[1]
  res = esn_state.shape[1]
  out_dim = W2.shape[1]

  npad = ((n + 1 + 2047) // 2048) * 2048
  rows = -(-e // 128)
  rw = -(-rows // (NW * CR)) * CR
  epad = NW * rw * 128

  row = jnp.concatenate(
      [edge_index[0], jnp.zeros((epad - e,), jnp.int32)]).reshape(NW * rw, 128)
  col = jnp.concatenate(
      [edge_index[1], jnp.full((epad - e,), n, jnp.int32)]).reshape(NW * rw, 128)
  x_pad = jnp.pad(node_features, ((0, npad - n), (0, 0)))
  zeros1 = jnp.zeros((npad,), jnp.float32)
  zeros2 = jnp.zeros((npad, 2), jnp.float32)
  ones2d = jnp.ones((1, 128), jnp.float32)

  hist = _sc_hist(npad, rw)(col, zeros1, ones2d)
  s, dinv = _tc_prep(npad, 2048)(hist.reshape(NC, npad, 1), x_pad)
  acc = _sc_scatter(npad, rw)(row, col, s, zeros2)

  blk = 4000
  return _tc_final(n, blk, hid, res, out_dim)(
      acc, s, dinv, esn_state,
      W_gcn, b_gcn.reshape(1, hid), ln_w.reshape(1, hid),
      ln_b.reshape(1, hid), W1[:hid], W1[hid:], b1.reshape(1, hid),
      W2, b2.reshape(1, out_dim))


# speedup vs baseline: 92.7399x; 2.4581x over previous
import functools

import jax
import jax.numpy as jnp
from jax import lax
from jax.experimental import pallas as pl
from jax.experimental.pallas import tpu as pltpu
from jax.experimental.pallas import tpu_sc as plsc

NC = 2
NS = 16
NW = NC * NS
CR = 30


def _sc_hist(npad, rows):
  mesh = plsc.VectorSubcoreMesh(core_axis_name="c", subcore_axis_name="s")
  rb = rows // NW
  ex = rows % NW
  nchunks = rb // CR

  @functools.partial(
      pl.kernel,
      out_type=jax.ShapeDtypeStruct((NC, NS, npad), jnp.float32),
      mesh=mesh,
      compiler_params=pltpu.CompilerParams(use_tc_tiling_on_sc=False,
                                           needs_layout_passes=False),
      scratch_types=[
          pltpu.VMEM((CR, 128), jnp.int32),
          pltpu.VMEM((npad,), jnp.float32),
      ],
  )
  def k(edge_hbm, zeros_hbm, out_hbm, idx_v, hist_v):
    cid = lax.axis_index("c")
    sid = lax.axis_index("s")
    wid = sid * NC + cid
    start = wid * rb + jnp.minimum(wid, ex)
    extra = (wid < ex).astype(jnp.int32)
    ones16 = jnp.ones((16,), jnp.float32)
    pltpu.sync_copy(zeros_hbm, hist_v)

    @pl.loop(0, nchunks)
    def _(t):
      pltpu.sync_copy(edge_hbm.at[1, pl.ds(start + t * CR, CR)], idx_v)

      @pl.loop(0, CR)
      def _(j):
        for i in range(8):
          plsc.addupdate_scatter(
              hist_v, [idx_v[j, pl.ds(i * 16, 16)]], ones16)

    @pl.loop(start + nchunks * CR, start + rb + extra)
    def _(r):
      pltpu.sync_copy(edge_hbm.at[1, pl.ds(r, 1)], idx_v.at[pl.ds(0, 1)])
      for i in range(8):
        plsc.addupdate_scatter(
            hist_v, [idx_v[0, pl.ds(i * 16, 16)]], ones16)

    pltpu.sync_copy(hist_v, out_hbm.at[cid, sid])

  return k


def _sc_scatter(npad, rows):
  mesh = plsc.VectorSubcoreMesh(core_axis_name="c", subcore_axis_name="s")
  rb = rows // NW
  ex = rows % NW
  nchunks = rb // CR

  @functools.partial(
      pl.kernel,
      out_type=jax.ShapeDtypeStruct((NC, NS, 2, npad), jnp.float32),
      mesh=mesh,
      compiler_params=pltpu.CompilerParams(use_tc_tiling_on_sc=False,
                                           needs_layout_passes=False),
      scratch_types=[
          pltpu.VMEM((CR, 128), jnp.int32),
          pltpu.VMEM((CR, 128), jnp.int32),
          pltpu.VMEM((CR, 128), jnp.float32),
          pltpu.VMEM((npad,), jnp.float32),
          pltpu.SemaphoreType.DMA,
      ],
  )
  def k(edge_hbm, s_hbm, zeros_hbm, out_hbm,
        row_v, col_v, val_v, acc_v, semg):
    cid = lax.axis_index("c")
    sid = lax.axis_index("s")
    wid = sid * NC + cid
    start = wid * rb + jnp.minimum(wid, ex)
    extra = (wid < ex).astype(jnp.int32)

    for plane in range(2):
      pltpu.sync_copy(zeros_hbm, acc_v)

      @pl.loop(0, nchunks)
      def _(t):
        pltpu.sync_copy(edge_hbm.at[0, pl.ds(start + t * CR, CR)], row_v)
        pltpu.sync_copy(edge_hbm.at[1, pl.ds(start + t * CR, CR)], col_v)

        @pl.loop(0, CR)
        def _(j):
          pltpu.async_copy(s_hbm.at[plane].at[row_v.at[j]], val_v.at[j],
                           semg)

        @pl.loop(0, CR)
        def _(j):
          pltpu.make_async_copy(s_hbm.at[plane].at[row_v.at[j]],
                                val_v.at[j], semg).wait()

        @pl.loop(0, CR)
        def _(j):
          for i in range(8):
            plsc.addupdate_scatter(
                acc_v, [col_v[j, pl.ds(i * 16, 16)]],
                val_v[j, pl.ds(i * 16, 16)])

      @pl.loop(start + nchunks * CR, start + rb + extra)
      def _(r):
        pltpu.sync_copy(edge_hbm.at[0, pl.ds(r, 1)], row_v.at[pl.ds(0, 1)])
        pltpu.sync_copy(edge_hbm.at[1, pl.ds(r, 1)], col_v.at[pl.ds(0, 1)])
        pltpu.sync_copy(s_hbm.at[plane].at[row_v.at[0]], val_v.at[0])
        for i in range(8):
          plsc.addupdate_scatter(
              acc_v, [col_v[0, pl.ds(i * 16, 16)]],
              val_v[0, pl.ds(i * 16, 16)])

      pltpu.sync_copy(acc_v, out_hbm.at[cid, sid, plane])

  return k


def _tc_prep(npad):
  m = npad // 128

  def body(h_ref, x_ref, s_ref, d_ref):
    deg = jnp.sum(h_ref[...], axis=(0, 1)) + 1.0
    dinv = 1.0 / jnp.sqrt(deg)
    d_ref[...] = dinv
    s_ref[0] = x_ref[0] * dinv
    s_ref[1] = x_ref[1] * dinv

  return pl.pallas_call(
      body,
      grid=(1,),
      in_specs=[
          pl.BlockSpec((NC, NS, m, 128), lambda i: (0, 0, 0, 0)),
          pl.BlockSpec((2, m, 128), lambda i: (0, 0, 0)),
      ],
      out_specs=[
          pl.BlockSpec((2, m, 128), lambda i: (0, 0, 0)),
          pl.BlockSpec((m, 128), lambda i: (0, 0)),
      ],
      out_shape=[
          jax.ShapeDtypeStruct((2, m, 128), jnp.float32),
          jax.ShapeDtypeStruct((m, 128), jnp.float32),
      ],
  )


def _tc_final(n, npad, br, hid, res, out_dim):
  m = npad // 128
  grid = m // br

  def body(acc_ref, s_ref, d_ref, esn_ref, wg_ref, bg_ref, lnw_ref, lnb_ref,
           w1a_ref, w1b_ref, b1_ref, w2_ref, b2_ref, o_ref, g_acc):
    i = pl.program_id(0)

    @pl.when(i == 0)
    def _():
      g_acc[...] = jnp.zeros_like(g_acc)

    acc = jnp.sum(acc_ref[...], axis=(0, 1))
    d = d_ref[...]
    a0 = (acc[0] + s_ref[0]) * d
    a1 = (acc[1] + s_ref[1]) * d
    w0 = wg_ref[...][0].reshape(hid, 1, 1)
    w1 = wg_ref[...][1].reshape(hid, 1, 1)
    bg = bg_ref[...].reshape(hid, 1, 1)
    h = w0 * a0[None] + w1 * a1[None] + bg
    h = jnp.maximum(h, 0.0)
    mu = jnp.mean(h, axis=0, keepdims=True)
    hc = h - mu
    var = jnp.mean(hc * hc, axis=0, keepdims=True)
    normed = (hc / jnp.sqrt(var + 1e-5) * lnw_ref[...].reshape(hid, 1, 1)
              + lnb_ref[...].reshape(hid, 1, 1))
    node = ((i * br) * 128
            + lax.broadcasted_iota(jnp.int32, (br, 128), 0) * 128
            + lax.broadcasted_iota(jnp.int32, (br, 128), 1))
    normed = jnp.where((node < n)[None], normed, 0.0)
    g_acc[...] += jnp.sum(normed, axis=(1, 2)).reshape(hid, 1)

    @pl.when(i == grid - 1)
    def _():
      g = g_acc[...].reshape(1, hid)
      z = (jnp.dot(g, w1a_ref[...], preferred_element_type=jnp.float32,
                   precision=lax.Precision.HIGHEST)
           + jnp.dot(esn_ref[...], w1b_ref[...],
                     preferred_element_type=jnp.float32,
                     precision=lax.Precision.HIGHEST)
           + b1_ref[...])
      z = jnp.maximum(z, 0.0)
      logits = jnp.dot(z, w2_ref[...], preferred_element_type=jnp.float32,
                       precision=lax.Precision.HIGHEST) + b2_ref[...]
      mx = jnp.max(logits, axis=1, keepdims=True)
      lse = jnp.log(jnp.sum(jnp.exp(logits - mx), axis=1, keepdims=True)) + mx
      o_ref[...] = logits - lse

  z0 = lambda i: (0, 0)
  return pl.pallas_call(
      body,
      grid=(grid,),
      in_specs=[
          pl.BlockSpec((NC, NS, 2, br, 128), lambda i: (0, 0, 0, i, 0)),
          pl.BlockSpec((2, br, 128), lambda i: (0, i, 0)),
          pl.BlockSpec((br, 128), lambda i: (i, 0)),
          pl.BlockSpec((1, res), z0),
          pl.BlockSpec((2, hid), z0),
          pl.BlockSpec((1, hid), z0),
          pl.BlockSpec((1, hid), z0),
          pl.BlockSpec((1, hid), z0),
          pl.BlockSpec((hid, hid), z0),
          pl.BlockSpec((res, hid), z0),
          pl.BlockSpec((1, hid), z0),
          pl.BlockSpec((hid, out_dim), z0),
          pl.BlockSpec((1, out_dim), z0),
      ],
      out_specs=pl.BlockSpec((1, out_dim), z0),
      out_shape=jax.ShapeDtypeStruct((1, out_dim), jnp.float32),
      scratch_shapes=[pltpu.VMEM((128, 1), jnp.float32)],
  )


def kernel(node_features, edge_index, esn_state, W_gcn, b_gcn, ln_w, ln_b,
           W1, b1, W2, b2):
  n, _ = node_features.shape
  e = edge_index.shape[1]
  hid = W_gcn.shape[1]
  res = esn_state.shape[1]
  out_dim = W2.shape[1]

  npad = ((n + 1 + 2047) // 2048) * 2048
  rows = e // 128
  if e % 128:
    pad = 128 - e % 128
    edge_index = jnp.concatenate(
        [edge_index,
         jnp.concatenate([jnp.zeros((1, pad), jnp.int32),
                          jnp.full((1, pad), n, jnp.int32)])], axis=1)
    rows += 1
  edge3 = edge_index.reshape(2, rows, 128)

  m = npad // 128
  xt = jnp.pad(node_features, ((0, npad - n), (0, 0))).T.reshape(2, m, 128)
  zeros1 = jnp.zeros((npad,), jnp.float32)

  hist = _sc_hist(npad, rows)(edge3, zeros1)
  s, dinv = _tc_prep(npad)(hist.reshape(NC, NS, m, 128), xt)
  s2 = s.reshape(2, npad)
  acc = _sc_scatter(npad, rows)(edge3, s2, zeros1)

  br = 112
  return _tc_final(n, npad, br, hid, res, out_dim)(
      acc.reshape(NC, NS, 2, m, 128), s, dinv, esn_state,
      W_gcn, b_gcn.reshape(1, hid), ln_w.reshape(1, hid),
      ln_b.reshape(1, hid), W1[:hid], W1[hid:], b1.reshape(1, hid),
      W2, b2.reshape(1, out_dim))


# speedup vs baseline: 102.5609x; 1.1059x over previous
import functools

import jax
import jax.numpy as jnp
from jax import lax
from jax.experimental import pallas as pl
from jax.experimental.pallas import tpu as pltpu
from jax.experimental.pallas import tpu_sc as plsc

NC = 2
NS = 16
NW = NC * NS
CR = 15


def _sc_hist(npad, rows):
  mesh = plsc.VectorSubcoreMesh(core_axis_name="c", subcore_axis_name="s")
  rb = rows // NW
  ex = rows % NW
  npairs = rb // (2 * CR)

  @functools.partial(
      pl.kernel,
      out_type=jax.ShapeDtypeStruct((NC, NS, npad), jnp.float32),
      mesh=mesh,
      compiler_params=pltpu.CompilerParams(use_tc_tiling_on_sc=False,
                                           needs_layout_passes=False),
      scratch_types=[
          pltpu.VMEM((CR, 128), jnp.int32),
          pltpu.VMEM((CR, 128), jnp.int32),
          pltpu.VMEM((npad,), jnp.float32),
          pltpu.SemaphoreType.DMA,
          pltpu.SemaphoreType.DMA,
      ],
  )
  def k(edge_hbm, zeros_hbm, out_hbm, idx_a, idx_b, hist_v, sem_a, sem_b):
    cid = lax.axis_index("c")
    sid = lax.axis_index("s")
    wid = sid * NC + cid
    start = wid * rb + jnp.minimum(wid, ex)
    extra = (wid < ex).astype(jnp.int32)
    ones16 = jnp.ones((16,), jnp.float32)
    pltpu.sync_copy(zeros_hbm, hist_v)

    def scat(buf):
      @pl.loop(0, CR)
      def _(j):
        for i in range(8):
          plsc.addupdate_scatter(
              hist_v, [buf[j, pl.ds(i * 16, 16)]], ones16)

    @pl.loop(0, npairs)
    def _(t):
      base = start + t * 2 * CR
      cp_a = pltpu.async_copy(edge_hbm.at[1, pl.ds(base, CR)], idx_a, sem_a)
      cp_b = pltpu.async_copy(edge_hbm.at[1, pl.ds(base + CR, CR)], idx_b,
                              sem_b)
      cp_a.wait()
      scat(idx_a)
      cp_b.wait()
      scat(idx_b)

    @pl.loop(start + npairs * 2 * CR, start + rb + extra)
    def _(r):
      pltpu.sync_copy(edge_hbm.at[1, pl.ds(r, 1)], idx_a.at[pl.ds(0, 1)])
      for i in range(8):
        plsc.addupdate_scatter(
            hist_v, [idx_a[0, pl.ds(i * 16, 16)]], ones16)

    pltpu.sync_copy(hist_v, out_hbm.at[cid, sid])

  return k


def _sc_scatter(npad, rows):
  mesh = plsc.VectorSubcoreMesh(core_axis_name="c", subcore_axis_name="s")
  rb = rows // NW
  ex = rows % NW
  npairs = rb // (2 * CR)

  @functools.partial(
      pl.kernel,
      out_type=jax.ShapeDtypeStruct((NC, NS, 2, npad), jnp.float32),
      mesh=mesh,
      compiler_params=pltpu.CompilerParams(use_tc_tiling_on_sc=False,
                                           needs_layout_passes=False),
      scratch_types=[
          pltpu.VMEM((CR, 128), jnp.int32),
          pltpu.VMEM((CR, 128), jnp.int32),
          pltpu.VMEM((CR, 128), jnp.float32),
          pltpu.VMEM((CR, 128), jnp.int32),
          pltpu.VMEM((CR, 128), jnp.int32),
          pltpu.VMEM((CR, 128), jnp.float32),
          pltpu.VMEM((npad,), jnp.float32),
          pltpu.SemaphoreType.DMA,
          pltpu.SemaphoreType.DMA,
      ],
  )
  def k(edge_hbm, s_hbm, zeros_hbm, out_hbm,
        row_a, col_a, val_a, row_b, col_b, val_b, acc_v, sem_a, sem_b):
    cid = lax.axis_index("c")
    sid = lax.axis_index("s")
    wid = sid * NC + cid
    start = wid * rb + jnp.minimum(wid, ex)
    extra = (wid < ex).astype(jnp.int32)

    for plane in range(2):
      pltpu.sync_copy(zeros_hbm, acc_v)
      splane = s_hbm.at[plane]

      def fire(row_v, val_v, sem):
        @pl.loop(0, CR)
        def _(j):
          pltpu.async_copy(splane.at[row_v.at[j]], val_v.at[j], sem)

      def drain(row_v, val_v, sem):
        @pl.loop(0, CR)
        def _(j):
          pltpu.make_async_copy(splane.at[row_v.at[j]], val_v.at[j],
                                sem).wait()

      def scat(col_v, val_v):
        @pl.loop(0, CR)
        def _(j):
          for i in range(8):
            plsc.addupdate_scatter(
                acc_v, [col_v[j, pl.ds(i * 16, 16)]],
                val_v[j, pl.ds(i * 16, 16)])

      @pl.loop(0, npairs)
      def _(t):
        base = start + t * 2 * CR
        pltpu.sync_copy(edge_hbm.at[0, pl.ds(base, CR)], row_a)
        pltpu.sync_copy(edge_hbm.at[1, pl.ds(base, CR)], col_a)
        fire(row_a, val_a, sem_a)
        pltpu.sync_copy(edge_hbm.at[0, pl.ds(base + CR, CR)], row_b)
        pltpu.sync_copy(edge_hbm.at[1, pl.ds(base + CR, CR)], col_b)
        fire(row_b, val_b, sem_b)
        drain(row_a, val_a, sem_a)
        scat(col_a, val_a)
        drain(row_b, val_b, sem_b)
        scat(col_b, val_b)

      @pl.loop(start + npairs * 2 * CR, start + rb + extra)
      def _(r):
        pltpu.sync_copy(edge_hbm.at[0, pl.ds(r, 1)], row_a.at[pl.ds(0, 1)])
        pltpu.sync_copy(edge_hbm.at[1, pl.ds(r, 1)], col_a.at[pl.ds(0, 1)])
        pltpu.sync_copy(splane.at[row_a.at[0]], val_a.at[0])
        for i in range(8):
          plsc.addupdate_scatter(
              acc_v, [col_a[0, pl.ds(i * 16, 16)]],
              val_a[0, pl.ds(i * 16, 16)])

      pltpu.sync_copy(acc_v, out_hbm.at[cid, sid, plane])

  return k


def _tc_prep(npad):
  m = npad // 128

  def body(h_ref, x_ref, s_ref, d_ref):
    deg = jnp.sum(h_ref[...], axis=(0, 1)) + 1.0
    dinv = 1.0 / jnp.sqrt(deg)
    d_ref[...] = dinv
    s_ref[0] = x_ref[0] * dinv
    s_ref[1] = x_ref[1] * dinv

  return pl.pallas_call(
      body,
      grid=(1,),
      in_specs=[
          pl.BlockSpec((NC, NS, m, 128), lambda i: (0, 0, 0, 0)),
          pl.BlockSpec((2, m, 128), lambda i: (0, 0, 0)),
      ],
      out_specs=[
          pl.BlockSpec((2, m, 128), lambda i: (0, 0, 0)),
          pl.BlockSpec((m, 128), lambda i: (0, 0)),
      ],
      out_shape=[
          jax.ShapeDtypeStruct((2, m, 128), jnp.float32),
          jax.ShapeDtypeStruct((m, 128), jnp.float32),
      ],
  )


def _tc_final(n, npad, br, hid, res, out_dim):
  m = npad // 128
  grid = m // br

  def body(acc_ref, s_ref, d_ref, esn_ref, wg_ref, bg_ref, lnw_ref, lnb_ref,
           w1a_ref, w1b_ref, b1_ref, w2_ref, b2_ref, o_ref, g_acc):
    i = pl.program_id(0)

    @pl.when(i == 0)
    def _():
      g_acc[...] = jnp.zeros_like(g_acc)

    acc = jnp.sum(acc_ref[...], axis=(0, 1))
    d = d_ref[...]
    a0 = (acc[0] + s_ref[0]) * d
    a1 = (acc[1] + s_ref[1]) * d
    w0 = wg_ref[...][0].reshape(hid, 1, 1)
    w1 = wg_ref[...][1].reshape(hid, 1, 1)
    bg = bg_ref[...].reshape(hid, 1, 1)
    h = w0 * a0[None] + w1 * a1[None] + bg
    h = jnp.maximum(h, 0.0)
    mu = jnp.mean(h, axis=0, keepdims=True)
    hc = h - mu
    var = jnp.mean(hc * hc, axis=0, keepdims=True)
    normed = (hc / jnp.sqrt(var + 1e-5) * lnw_ref[...].reshape(hid, 1, 1)
              + lnb_ref[...].reshape(hid, 1, 1))
    node = ((i * br) * 128
            + lax.broadcasted_iota(jnp.int32, (br, 128), 0) * 128
            + lax.broadcasted_iota(jnp.int32, (br, 128), 1))
    normed = jnp.where((node < n)[None], normed, 0.0)
    g_acc[...] += jnp.sum(normed, axis=(1, 2)).reshape(hid, 1)

    @pl.when(i == grid - 1)
    def _():
      g = g_acc[...].reshape(1, hid)
      z = (jnp.dot(g, w1a_ref[...], preferred_element_type=jnp.float32,
                   precision=lax.Precision.HIGHEST)
           + jnp.dot(esn_ref[...], w1b_ref[...],
                     preferred_element_type=jnp.float32,
                     precision=lax.Precision.HIGHEST)
           + b1_ref[...])
      z = jnp.maximum(z, 0.0)
      logits = jnp.dot(z, w2_ref[...], preferred_element_type=jnp.float32,
                       precision=lax.Precision.HIGHEST) + b2_ref[...]
      mx = jnp.max(logits, axis=1, keepdims=True)
      lse = jnp.log(jnp.sum(jnp.exp(logits - mx), axis=1, keepdims=True)) + mx
      o_ref[...] = logits - lse

  z0 = lambda i: (0, 0)
  return pl.pallas_call(
      body,
      grid=(grid,),
      in_specs=[
          pl.BlockSpec((NC, NS, 2, br, 128), lambda i: (0, 0, 0, i, 0)),
          pl.BlockSpec((2, br, 128), lambda i: (0, i, 0)),
          pl.BlockSpec((br, 128), lambda i: (i, 0)),
          pl.BlockSpec((1, res), z0),
          pl.BlockSpec((2, hid), z0),
          pl.BlockSpec((1, hid), z0),
          pl.BlockSpec((1, hid), z0),
          pl.BlockSpec((1, hid), z0),
          pl.BlockSpec((hid, hid), z0),
          pl.BlockSpec((res, hid), z0),
          pl.BlockSpec((1, hid), z0),
          pl.BlockSpec((hid, out_dim), z0),
          pl.BlockSpec((1, out_dim), z0),
      ],
      out_specs=pl.BlockSpec((1, out_dim), z0),
      out_shape=jax.ShapeDtypeStruct((1, out_dim), jnp.float32),
      scratch_shapes=[pltpu.VMEM((128, 1), jnp.float32)],
  )


def kernel(node_features, edge_index, esn_state, W_gcn, b_gcn, ln_w, ln_b,
           W1, b1, W2, b2):
  n, _ = node_features.shape
  e = edge_index.shape[1]
  hid = W_gcn.shape[1]
  res = esn_state.shape[1]
  out_dim = W2.shape[1]

  npad = ((n + 1 + 2047) // 2048) * 2048
  rows = e // 128
  if e % 128:
    pad = 128 - e % 128
    edge_index = jnp.concatenate(
        [edge_index,
         jnp.concatenate([jnp.zeros((1, pad), jnp.int32),
                          jnp.full((1, pad), n, jnp.int32)])], axis=1)
    rows += 1
  edge3 = edge_index.reshape(2, rows, 128)

  m = npad // 128
  xt = jnp.pad(node_features, ((0, npad - n), (0, 0))).T.reshape(2, m, 128)
  zeros1 = jnp.zeros((npad,), jnp.float32)

  hist = _sc_hist(npad, rows)(edge3, zeros1)
  s, dinv = _tc_prep(npad)(hist.reshape(NC, NS, m, 128), xt)
  s2 = s.reshape(2, npad)
  acc = _sc_scatter(npad, rows)(edge3, s2, zeros1)

  br = 112
  return _tc_final(n, npad, br, hid, res, out_dim)(
      acc.reshape(NC, NS, 2, m, 128), s, dinv, esn_state,
      W_gcn, b_gcn.reshape(1, hid), ln_w.reshape(1, hid),
      ln_b.reshape(1, hid), W1[:hid], W1[hid:], b1.reshape(1, hid),
      W2, b2.reshape(1, out_dim))


# speedup vs baseline: 105.7343x; 1.0309x over previous
import functools

import jax
import jax.numpy as jnp
from jax import lax
from jax.experimental import pallas as pl
from jax.experimental.pallas import tpu as pltpu
from jax.experimental.pallas import tpu_sc as plsc

NC = 2
NS = 16
NW = NC * NS
CR = 39


def _sc_hist(npad, rows):
  mesh = plsc.VectorSubcoreMesh(core_axis_name="c", subcore_axis_name="s")
  rb = rows // NW
  ex = rows % NW
  npairs = rb // (2 * CR)

  @functools.partial(
      pl.kernel,
      out_type=jax.ShapeDtypeStruct((NC, NS, npad), jnp.float32),
      mesh=mesh,
      compiler_params=pltpu.CompilerParams(use_tc_tiling_on_sc=False,
                                           needs_layout_passes=False),
      scratch_types=[
          pltpu.VMEM((CR, 128), jnp.int32),
          pltpu.VMEM((CR, 128), jnp.int32),
          pltpu.VMEM((npad,), jnp.float32),
          pltpu.SemaphoreType.DMA,
          pltpu.SemaphoreType.DMA,
      ],
  )
  def k(edge_hbm, zeros_hbm, out_hbm, idx_a, idx_b, hist_v, sem_a, sem_b):
    cid = lax.axis_index("c")
    sid = lax.axis_index("s")
    wid = sid * NC + cid
    start = wid * rb + jnp.minimum(wid, ex)
    extra = (wid < ex).astype(jnp.int32)
    ones16 = jnp.ones((16,), jnp.float32)
    pltpu.sync_copy(zeros_hbm, hist_v)

    def scat(buf):
      @pl.loop(0, CR)
      def _(j):
        for i in range(8):
          plsc.addupdate_scatter(
              hist_v, [buf[j, pl.ds(i * 16, 16)]], ones16)

    @pl.loop(0, npairs)
    def _(t):
      base = start + t * 2 * CR
      cp_a = pltpu.async_copy(edge_hbm.at[1, pl.ds(base, CR)], idx_a, sem_a)
      cp_b = pltpu.async_copy(edge_hbm.at[1, pl.ds(base + CR, CR)], idx_b,
                              sem_b)
      cp_a.wait()
      scat(idx_a)
      cp_b.wait()
      scat(idx_b)

    @pl.loop(start + npairs * 2 * CR, start + rb + extra)
    def _(r):
      pltpu.sync_copy(edge_hbm.at[1, pl.ds(r, 1)], idx_a.at[pl.ds(0, 1)])
      for i in range(8):
        plsc.addupdate_scatter(
            hist_v, [idx_a[0, pl.ds(i * 16, 16)]], ones16)

    pltpu.sync_copy(hist_v, out_hbm.at[cid, sid])

  return k


def _sc_scatter(npad, rows):
  mesh = plsc.VectorSubcoreMesh(core_axis_name="c", subcore_axis_name="s")
  rb = rows // NW
  ex = rows % NW
  npairs = rb // (2 * CR)

  @functools.partial(
      pl.kernel,
      out_type=jax.ShapeDtypeStruct((NC, NS, 2, npad), jnp.float32),
      mesh=mesh,
      compiler_params=pltpu.CompilerParams(use_tc_tiling_on_sc=False,
                                           needs_layout_passes=False),
      scratch_types=[
          pltpu.VMEM((CR, 128), jnp.int32),
          pltpu.VMEM((CR, 128), jnp.int32),
          pltpu.VMEM((CR, 128), jnp.float32),
          pltpu.VMEM((CR, 128), jnp.int32),
          pltpu.VMEM((CR, 128), jnp.int32),
          pltpu.VMEM((CR, 128), jnp.float32),
          pltpu.VMEM((npad,), jnp.float32),
          pltpu.SemaphoreType.DMA,
          pltpu.SemaphoreType.DMA,
      ],
  )
  def k(edge_hbm, s_hbm, zeros_hbm, out_hbm,
        row_a, col_a, val_a, row_b, col_b, val_b, acc_v, sem_a, sem_b):
    cid = lax.axis_index("c")
    sid = lax.axis_index("s")
    wid = sid * NC + cid
    start = wid * rb + jnp.minimum(wid, ex)
    extra = (wid < ex).astype(jnp.int32)

    for plane in range(2):
      pltpu.sync_copy(zeros_hbm, acc_v)
      splane = s_hbm.at[plane]

      def fire(row_v, val_v, sem):
        @pl.loop(0, CR)
        def _(j):
          pltpu.async_copy(splane.at[row_v.at[j]], val_v.at[j], sem)

      def drain(row_v, val_v, sem):
        @pl.loop(0, CR)
        def _(j):
          pltpu.make_async_copy(splane.at[row_v.at[j]], val_v.at[j],
                                sem).wait()

      def scat(col_v, val_v):
        @pl.loop(0, CR)
        def _(j):
          for i in range(8):
            plsc.addupdate_scatter(
                acc_v, [col_v[j, pl.ds(i * 16, 16)]],
                val_v[j, pl.ds(i * 16, 16)])

      @pl.loop(0, npairs)
      def _(t):
        base = start + t * 2 * CR
        pltpu.sync_copy(edge_hbm.at[0, pl.ds(base, CR)], row_a)
        pltpu.sync_copy(edge_hbm.at[1, pl.ds(base, CR)], col_a)
        fire(row_a, val_a, sem_a)
        pltpu.sync_copy(edge_hbm.at[0, pl.ds(base + CR, CR)], row_b)
        pltpu.sync_copy(edge_hbm.at[1, pl.ds(base + CR, CR)], col_b)
        fire(row_b, val_b, sem_b)
        drain(row_a, val_a, sem_a)
        scat(col_a, val_a)
        drain(row_b, val_b, sem_b)
        scat(col_b, val_b)

      @pl.loop(start + npairs * 2 * CR, start + rb + extra)
      def _(r):
        pltpu.sync_copy(edge_hbm.at[0, pl.ds(r, 1)], row_a.at[pl.ds(0, 1)])
        pltpu.sync_copy(edge_hbm.at[1, pl.ds(r, 1)], col_a.at[pl.ds(0, 1)])
        pltpu.sync_copy(splane.at[row_a.at[0]], val_a.at[0])
        for i in range(8):
          plsc.addupdate_scatter(
              acc_v, [col_a[0, pl.ds(i * 16, 16)]],
              val_a[0, pl.ds(i * 16, 16)])

      pltpu.sync_copy(acc_v, out_hbm.at[cid, sid, plane])

  return k


def _tc_prep(npad):
  m = npad // 128

  def body(h_ref, x_ref, s_ref, d_ref):
    deg = jnp.sum(h_ref[...], axis=(0, 1)) + 1.0
    dinv = 1.0 / jnp.sqrt(deg)
    d_ref[...] = dinv
    s_ref[0] = x_ref[0] * dinv
    s_ref[1] = x_ref[1] * dinv

  return pl.pallas_call(
      body,
      grid=(1,),
      in_specs=[
          pl.BlockSpec((NC, NS, m, 128), lambda i: (0, 0, 0, 0)),
          pl.BlockSpec((2, m, 128), lambda i: (0, 0, 0)),
      ],
      out_specs=[
          pl.BlockSpec((2, m, 128), lambda i: (0, 0, 0)),
          pl.BlockSpec((m, 128), lambda i: (0, 0)),
      ],
      out_shape=[
          jax.ShapeDtypeStruct((2, m, 128), jnp.float32),
          jax.ShapeDtypeStruct((m, 128), jnp.float32),
      ],
  )


def _tc_final(n, npad, br, hid, res, out_dim):
  m = npad // 128
  grid = m // br

  def body(acc_ref, s_ref, d_ref, esn_ref, wg_ref, bg_ref, lnw_ref, lnb_ref,
           w1a_ref, w1b_ref, b1_ref, w2_ref, b2_ref, o_ref, g_acc):
    i = pl.program_id(0)

    @pl.when(i == 0)
    def _():
      g_acc[...] = jnp.zeros_like(g_acc)

    acc = jnp.sum(acc_ref[...], axis=(0, 1))
    d = d_ref[...]
    a0 = (acc[0] + s_ref[0]) * d
    a1 = (acc[1] + s_ref[1]) * d
    w0 = wg_ref[...][0].reshape(hid, 1, 1)
    w1 = wg_ref[...][1].reshape(hid, 1, 1)
    bg = bg_ref[...].reshape(hid, 1, 1)
    h = w0 * a0[None] + w1 * a1[None] + bg
    h = jnp.maximum(h, 0.0)
    mu = jnp.mean(h, axis=0, keepdims=True)
    hc = h - mu
    var = jnp.mean(hc * hc, axis=0, keepdims=True)
    normed = (hc / jnp.sqrt(var + 1e-5) * lnw_ref[...].reshape(hid, 1, 1)
              + lnb_ref[...].reshape(hid, 1, 1))
    node = ((i * br) * 128
            + lax.broadcasted_iota(jnp.int32, (br, 128), 0) * 128
            + lax.broadcasted_iota(jnp.int32, (br, 128), 1))
    normed = jnp.where((node < n)[None], normed, 0.0)
    g_acc[...] += jnp.sum(normed, axis=(1, 2)).reshape(hid, 1)

    @pl.when(i == grid - 1)
    def _():
      g = g_acc[...].reshape(1, hid)
      z = (jnp.dot(g, w1a_ref[...], preferred_element_type=jnp.float32,
                   precision=lax.Precision.HIGHEST)
           + jnp.dot(esn_ref[...], w1b_ref[...],
                     preferred_element_type=jnp.float32,
                     precision=lax.Precision.HIGHEST)
           + b1_ref[...])
      z = jnp.maximum(z, 0.0)
      logits = jnp.dot(z, w2_ref[...], preferred_element_type=jnp.float32,
                       precision=lax.Precision.HIGHEST) + b2_ref[...]
      mx = jnp.max(logits, axis=1, keepdims=True)
      lse = jnp.log(jnp.sum(jnp.exp(logits - mx), axis=1, keepdims=True)) + mx
      o_ref[...] = logits - lse

  z0 = lambda i: (0, 0)
  return pl.pallas_call(
      body,
      grid=(grid,),
      in_specs=[
          pl.BlockSpec((NC, NS, 2, br, 128), lambda i: (0, 0, 0, i, 0)),
          pl.BlockSpec((2, br, 128), lambda i: (0, i, 0)),
          pl.BlockSpec((br, 128), lambda i: (i, 0)),
          pl.BlockSpec((1, res), z0),
          pl.BlockSpec((2, hid), z0),
          pl.BlockSpec((1, hid), z0),
          pl.BlockSpec((1, hid), z0),
          pl.BlockSpec((1, hid), z0),
          pl.BlockSpec((hid, hid), z0),
          pl.BlockSpec((res, hid), z0),
          pl.BlockSpec((1, hid), z0),
          pl.BlockSpec((hid, out_dim), z0),
          pl.BlockSpec((1, out_dim), z0),
      ],
      out_specs=pl.BlockSpec((1, out_dim), z0),
      out_shape=jax.ShapeDtypeStruct((1, out_dim), jnp.float32),
      scratch_shapes=[pltpu.VMEM((128, 1), jnp.float32)],
  )


def kernel(node_features, edge_index, esn_state, W_gcn, b_gcn, ln_w, ln_b,
           W1, b1, W2, b2):
  n, _ = node_features.shape
  e = edge_index.shape[1]
  hid = W_gcn.shape[1]
  res = esn_state.shape[1]
  out_dim = W2.shape[1]

  npad = ((n + 1 + 2047) // 2048) * 2048
  rows = e // 128
  if e % 128:
    pad = 128 - e % 128
    edge_index = jnp.concatenate(
        [edge_index,
         jnp.concatenate([jnp.zeros((1, pad), jnp.int32),
                          jnp.full((1, pad), n, jnp.int32)])], axis=1)
    rows += 1
  edge3 = edge_index.reshape(2, rows, 128)

  m = npad // 128
  zeros1 = jnp.zeros((npad,), jnp.float32)

  hist = _sc_hist(npad, rows)(edge3, zeros1)
  xt = jnp.pad(node_features, ((0, npad - n), (0, 0))).T.reshape(2, m, 128)
  s, dinv = _tc_prep(npad)(hist.reshape(NC, NS, m, 128), xt)
  s2 = s.reshape(2, npad)
  acc = _sc_scatter(npad, rows)(edge3, s2, zeros1)

  br = 112
  return _tc_final(n, npad, br, hid, res, out_dim)(
      acc.reshape(NC, NS, 2, m, 128), s, dinv, esn_state,
      W_gcn, b_gcn.reshape(1, hid), ln_w.reshape(1, hid),
      ln_b.reshape(1, hid), W1[:hid], W1[hid:], b1.reshape(1, hid),
      W2, b2.reshape(1, out_dim))
